# baseline (speedup 1.0000x reference)
import jax
import jax.numpy as jnp
from jax import lax
from jax.experimental import pallas as pl
from jax.experimental.pallas import tpu as pltpu
from jax.experimental.pallas import tpu_sc as plsc

WINDOW = 128


NCHUNK = 4


def kernel(atomic_numbers, embedding_table):
    B, S = atomic_numbers.shape
    V, D = embedding_table.shape
    CB = B // NCHUNK

    mesh = plsc.VectorSubcoreMesh(core_axis_name="c", subcore_axis_name="s")

    BLK_B = 4

    @pl.kernel(
        out_type=jax.ShapeDtypeStruct((CB, S, D), embedding_table.dtype),
        mesh=mesh,
        scratch_types=[
            pltpu.VMEM_SHARED((V, D), embedding_table.dtype),
            pltpu.SemaphoreType.DMA,
        ],
    )
    def gather_kernel(table_hbm, idx_hbm, out_hbm, table_spmem, sem):
        @pl.when(lax.axis_index("s") == 0)
        def _():
            pltpu.sync_copy(table_hbm, table_spmem)

        plsc.subcore_barrier()

        def body(i_vmem, o_vmem):
            copies = [
                pltpu.async_copy(
                    table_spmem.at[i_vmem.at[k]],
                    o_vmem.at[k],
                    sem,
                )
                for k in range(BLK_B)
            ]
            for c in copies:
                c.wait()

        pltpu.emit_pipeline(
            body,
            grid=(CB // BLK_B,),
            in_specs=[pl.BlockSpec((BLK_B, S), index_map=lambda i: (i, 0))],
            out_specs=[
                pl.BlockSpec((BLK_B, S, D), index_map=lambda i: (i, 0, 0))
            ],
            core_axis_name=("c", "s"),
            dimension_semantics=(pltpu.PARALLEL,),
        )(idx_hbm, out_hbm)

    chunks = [
        gather_kernel(embedding_table, atomic_numbers[c * CB:(c + 1) * CB])
        for c in range(NCHUNK)
    ]
    return jnp.concatenate(chunks, axis=0)


# speedup vs baseline: 1.0456x; 1.0456x over previous
import jax
import jax.numpy as jnp
from jax import lax
from jax.experimental import pallas as pl
from jax.experimental.pallas import tpu as pltpu
from jax.experimental.pallas import tpu_sc as plsc

WINDOW = 128


NCHUNK = 4


def kernel(atomic_numbers, embedding_table):
    B, S = atomic_numbers.shape
    V, D = embedding_table.shape
    CB = B // NCHUNK

    mesh = plsc.VectorSubcoreMesh(core_axis_name="c", subcore_axis_name="s")

    BLK_B = 4

    @pl.kernel(
        out_type=jax.ShapeDtypeStruct((CB, S, D), embedding_table.dtype),
        mesh=mesh,
        scratch_types=[
            pltpu.VMEM_SHARED((V, D), embedding_table.dtype),
            pltpu.SemaphoreType.DMA,
        ],
    )
    def gather_kernel(table_hbm, idx_hbm, out_hbm, table_spmem, sem):
        @pl.when(lax.axis_index("s") == 0)
        def _():
            pltpu.sync_copy(table_hbm, table_spmem)

        plsc.subcore_barrier()

        def body(i_vmem, o_vmem):
            copies = [
                pltpu.async_copy(
                    table_spmem.at[i_vmem.at[k]],
                    o_vmem.at[k],
                    sem,
                )
                for k in range(BLK_B)
            ]
            for c in copies:
                c.wait()

        pltpu.emit_pipeline(
            body,
            grid=(CB // BLK_B,),
            in_specs=[pl.BlockSpec((BLK_B, S), index_map=lambda i: (i, 0))],
            out_specs=[
                pl.BlockSpec((BLK_B, S, D), index_map=lambda i: (i, 0, 0))
            ],
            core_axis_name=("c", "s"),
            dimension_semantics=(pltpu.PARALLEL,),
        )(idx_hbm, out_hbm)

    out = jnp.zeros((B, S, D), embedding_table.dtype)
    for c in range(NCHUNK):
        chunk = gather_kernel(
            embedding_table, atomic_numbers[c * CB:(c + 1) * CB]
        )
        out = jax.lax.dynamic_update_slice(out, chunk, (c * CB, 0, 0))
    return out
